# baseline (speedup 1.0000x reference)
import functools

import jax
import jax.numpy as jnp
from jax import lax
from jax.experimental import pallas as pl
from jax.experimental.pallas import tpu as pltpu
from jax.experimental.pallas import tpu_sc as plsc

NUM_EMB = 1000000
H = 64
BATCH = 4096
HIST = 200

NC = 2
NS = 16
NW = NC * NS

N = BATCH * HIST
M = N // NW
C = 128
K = M // C


def _make_gather():
    mesh = plsc.VectorSubcoreMesh(core_axis_name="c", subcore_axis_name="s")

    @functools.partial(
        pl.kernel,
        mesh=mesh,
        out_type=jax.ShapeDtypeStruct((N, H), jnp.float32),
        scratch_types=[
            pltpu.VMEM((M,), jnp.int32),
            pltpu.VMEM((C, H), jnp.float32),
            pltpu.SemaphoreType.DMA,
        ],
        compiler_params=pltpu.CompilerParams(use_tc_tiling_on_sc=False),
    )
    def k(table_hbm, idx_hbm, out_hbm, idx_v, rows_v, gsem):
        wid = lax.axis_index("s") * NC + lax.axis_index("c")
        base = wid * M
        pltpu.sync_copy(idx_hbm.at[pl.ds(base, M)], idx_v)

        def chunk(j, carry):
            idx_slice = idx_v.at[pl.ds(j * C, C)]
            pltpu.async_copy(table_hbm.at[idx_slice], rows_v, gsem).wait()
            pltpu.sync_copy(rows_v, out_hbm.at[pl.ds(base + j * C, C)])
            return carry

        lax.fori_loop(0, K, chunk, 0)

    return k


_gather = _make_gather()


@jax.jit
def kernel(input_ids, table):
    ids_flat = input_ids.reshape(-1).astype(jnp.int32)
    out = _gather(table, ids_flat)
    return out.reshape(BATCH, HIST, H)


# speedup vs baseline: 1.1149x; 1.1149x over previous
import functools

import jax
import jax.numpy as jnp
from jax import lax
from jax.experimental import pallas as pl
from jax.experimental.pallas import tpu as pltpu
from jax.experimental.pallas import tpu_sc as plsc

NUM_EMB = 1000000
H = 64
BATCH = 4096
HIST = 200

NC = 2
NS = 16
NW = NC * NS

N = BATCH * HIST
M = N // NW
C = 128
K = M // C
NBUF = 8
LAG = 4
T = K // NBUF


def _make_gather():
    mesh = plsc.VectorSubcoreMesh(core_axis_name="c", subcore_axis_name="s")

    @functools.partial(
        pl.kernel,
        mesh=mesh,
        out_type=jax.ShapeDtypeStruct((N, H), jnp.float32),
        scratch_types=[
            pltpu.VMEM((M,), jnp.int32),
            pltpu.VMEM((NBUF, C, H), jnp.float32),
            pltpu.SemaphoreType.DMA((NBUF,)),
            pltpu.SemaphoreType.DMA((NBUF,)),
        ],
        compiler_params=pltpu.CompilerParams(use_tc_tiling_on_sc=False),
    )
    def k(table_hbm, idx_hbm, out_hbm, idx_v, rows_v, gsem, osem):
        wid = lax.axis_index("s") * NC + lax.axis_index("c")
        base = wid * M
        pltpu.sync_copy(idx_hbm.at[pl.ds(base, M)], idx_v)

        def gather_desc(j, slot):
            return pltpu.make_async_copy(
                table_hbm.at[idx_v.at[pl.ds(j * C, C)]],
                rows_v.at[slot],
                gsem.at[slot],
            )

        def store_desc(j, slot):
            return pltpu.make_async_copy(
                rows_v.at[slot],
                out_hbm.at[pl.ds(base + j * C, C)],
                osem.at[slot],
            )

        def round_body(t, carry):
            for b in range(NBUF):
                j = t * NBUF + b
                @pl.when(j >= NBUF)
                def _():
                    store_desc(j - NBUF, b).wait()

                gather_desc(j, b).start()

                j2 = j - LAG
                b2 = (b + NBUF - LAG) % NBUF

                @pl.when(j2 >= 0)
                def _():
                    gather_desc(j2, b2).wait()
                    store_desc(j2, b2).start()

            return carry

        lax.fori_loop(0, T, round_body, 0)

        for b in range(NBUF - LAG, NBUF):
            j2 = K - NBUF + b
            gather_desc(j2, b).wait()
            store_desc(j2, b).start()
        for b in range(NBUF):
            store_desc(K - NBUF + b, b).wait()

    return k


_gather = _make_gather()


@jax.jit
def kernel(input_ids, table):
    ids_flat = input_ids.reshape(-1).astype(jnp.int32)
    out = _gather(table, ids_flat)
    return out.reshape(BATCH, HIST, H)


# speedup vs baseline: 1.1163x; 1.0013x over previous
import functools

import jax
import jax.numpy as jnp
from jax import lax
from jax.experimental import pallas as pl
from jax.experimental.pallas import tpu as pltpu
from jax.experimental.pallas import tpu_sc as plsc

NUM_EMB = 1000000
H = 64
BATCH = 4096
HIST = 200

NC = 2
NS = 16
NW = NC * NS

N = BATCH * HIST
M = N // NW
C = 256
K = M // C
NBUF = 4
LAG = 2
T = K // NBUF


def _make_gather():
    mesh = plsc.VectorSubcoreMesh(core_axis_name="c", subcore_axis_name="s")

    @functools.partial(
        pl.kernel,
        mesh=mesh,
        out_type=jax.ShapeDtypeStruct((N, H), jnp.float32),
        scratch_types=[
            pltpu.VMEM((M,), jnp.int32),
            pltpu.VMEM((NBUF, C, H), jnp.float32),
            pltpu.SemaphoreType.DMA((NBUF,)),
            pltpu.SemaphoreType.DMA((NBUF,)),
        ],
        compiler_params=pltpu.CompilerParams(use_tc_tiling_on_sc=False),
    )
    def k(table_hbm, idx_hbm, out_hbm, idx_v, rows_v, gsem, osem):
        wid = lax.axis_index("s") * NC + lax.axis_index("c")
        base = wid * M
        pltpu.sync_copy(idx_hbm.at[pl.ds(base, M)], idx_v)

        def gather_desc(j, slot):
            return pltpu.make_async_copy(
                table_hbm.at[idx_v.at[pl.ds(j * C, C)]],
                rows_v.at[slot],
                gsem.at[slot],
            )

        def store_desc(j, slot):
            return pltpu.make_async_copy(
                rows_v.at[slot],
                out_hbm.at[pl.ds(base + j * C, C)],
                osem.at[slot],
            )

        def round_body(t, carry):
            for b in range(NBUF):
                j = t * NBUF + b
                @pl.when(j >= NBUF)
                def _():
                    store_desc(j - NBUF, b).wait()

                gather_desc(j, b).start()

                j2 = j - LAG
                b2 = (b + NBUF - LAG) % NBUF

                @pl.when(j2 >= 0)
                def _():
                    gather_desc(j2, b2).wait()
                    store_desc(j2, b2).start()

            return carry

        lax.fori_loop(0, T, round_body, 0)

        for b in range(NBUF - LAG, NBUF):
            j2 = K - NBUF + b
            gather_desc(j2, b).wait()
            store_desc(j2, b).start()
        for b in range(NBUF):
            store_desc(K - NBUF + b, b).wait()

    return k


_gather = _make_gather()


@jax.jit
def kernel(input_ids, table):
    ids_flat = input_ids.reshape(-1).astype(jnp.int32)
    out = _gather(table, ids_flat)
    return out.reshape(BATCH, HIST, H)
